# baseline (speedup 1.0000x reference)
import functools

import jax
import jax.numpy as jnp
from jax import lax
from jax.experimental import pallas as pl
from jax.experimental.pallas import tpu as pltpu
from jax.experimental.pallas import tpu_sc as plsc

BATCH = 16384
EMBED = 128
ROWS_PER_BLOCK = 2048
LANES = 16


def _score_body(h_ref, t_ref, r_ref, o_ref):
    a = jnp.abs(h_ref[...] + r_ref[...] - t_ref[...])
    p = a[:, 0:8]
    for k in range(1, 16):
        p = p + a[:, 8 * k:8 * k + 8]
    q = p[:, 0:4] + p[:, 4:8]
    r2 = q[:, 0:2] + q[:, 2:4]
    o_ref[...] = -(r2[:, 0] + r2[:, 1])


def _tc_score(ph, pt, pr):
    grid = BATCH // ROWS_PER_BLOCK
    spec = pl.BlockSpec((ROWS_PER_BLOCK, EMBED), lambda i: (i, 0))
    return pl.pallas_call(
        _score_body,
        grid=(grid,),
        in_specs=[spec, spec, spec],
        out_specs=pl.BlockSpec((ROWS_PER_BLOCK,), lambda i: (i,)),
        out_shape=jax.ShapeDtypeStruct((BATCH,), jnp.float32),
    )(ph, pt, pr)


def _sort_body(score_hbm, out_hbm, ka, va, kb, vb, hist, sc, sem):
    cid = lax.axis_index("c")
    sid = lax.axis_index("s")
    del sem
    NG = BATCH // LANES
    lanes = lax.iota(jnp.int32, LANES)

    @pl.when(jnp.logical_and(cid == 0, sid == 0))
    def _():
        pltpu.sync_copy(score_hbm, sc)

        def build(g, _):
            s = sc[pl.ds(g * LANES, LANES)]
            b = lax.bitcast_convert_type(-s, jnp.int32)
            key = jnp.where(b < 0, ~b, b | jnp.int32(-2**31))
            ka[pl.ds(g * LANES, LANES)] = key
            va[pl.ds(g * LANES, LANES)] = g * LANES + lanes
            return 0

        lax.fori_loop(0, NG, build, 0)

        eidx0 = lanes * NG
        for p, (sk, sv, dk, dv) in enumerate(
                ((ka, va, kb, vb), (kb, vb, ka, va),
                 (ka, va, kb, vb), (kb, vb, ka, va))):
            shift = jnp.full((LANES,), 8 * p, jnp.int32)

            def zero(i, _):
                hist[pl.ds(i * LANES, LANES)] = jnp.zeros((LANES,), jnp.int32)
                return 0

            lax.fori_loop(0, 256, zero, 0)

            def histo(g, _, sk=sk):
                key = plsc.load_gather(sk, [eidx0 + g])
                d = lax.shift_right_logical(key, shift) & 255
                idx = d * LANES + lanes
                c = plsc.load_gather(hist, [idx])
                plsc.store_scatter(hist, [idx], c + 1)
                return 0

            lax.fori_loop(0, NG, histo, 0)

            def scan(i, carry):
                v = hist[pl.ds(i * LANES, LANES)]
                inc = plsc.cumsum(v)
                hist[pl.ds(i * LANES, LANES)] = inc - v + carry
                return carry + jnp.sum(v)

            lax.fori_loop(0, 256, scan, jnp.int32(0))

            def perm(g, _, sk=sk, sv=sv, dk=dk, dv=dv):
                e = eidx0 + g
                key = plsc.load_gather(sk, [e])
                val = plsc.load_gather(sv, [e])
                d = lax.shift_right_logical(key, shift) & 255
                idx = d * LANES + lanes
                pos = plsc.load_gather(hist, [idx])
                plsc.store_scatter(hist, [idx], pos + 1)
                plsc.store_scatter(dk, [pos], key)
                plsc.store_scatter(dv, [pos], val)
                return 0

            lax.fori_loop(0, NG, perm, 0)
        pltpu.sync_copy(va, out_hbm)


def _sc_sort(p_score):
    mesh = plsc.VectorSubcoreMesh(core_axis_name="c", subcore_axis_name="s")
    return pl.kernel(
        _sort_body,
        mesh=mesh,
        compiler_params=pltpu.CompilerParams(needs_layout_passes=False),
        out_type=jax.ShapeDtypeStruct((BATCH,), jnp.int32),
        scratch_types=[
            pltpu.VMEM((BATCH,), jnp.int32),
            pltpu.VMEM((BATCH,), jnp.int32),
            pltpu.VMEM((BATCH,), jnp.int32),
            pltpu.VMEM((BATCH,), jnp.int32),
            pltpu.VMEM((256 * LANES,), jnp.int32),
            pltpu.VMEM((BATCH,), jnp.float32),
            pltpu.SemaphoreType.DMA,
        ],
    )(p_score)


def kernel(h_idx, t_idx, r_idx, ent_table, rel_table):
    ph = jnp.take(ent_table, h_idx, axis=0)
    pt = jnp.take(ent_table, t_idx, axis=0)
    pr = jnp.take(rel_table, r_idx, axis=0)
    p_score = _tc_score(ph, pt, pr)
    ranked = _sc_sort(p_score)
    return (p_score, ranked)


# speedup vs baseline: 4.0596x; 4.0596x over previous
import functools

import jax
import jax.numpy as jnp
from jax import lax
from jax.experimental import pallas as pl
from jax.experimental.pallas import tpu as pltpu
from jax.experimental.pallas import tpu_sc as plsc

BATCH = 16384
EMBED = 128
ROWS_PER_BLOCK = 2048
LANES = 16
_DBG = None


def _score_sc_body(h_hbm, t_hbm, r_hbm, ent_hbm, rel_hbm, ps_hbm,
                   hi, ti, ri, hb, tb2, rb, sb, sem):
    cid = lax.axis_index("c")
    sid = lax.axis_index("s")
    lanes = lax.iota(jnp.int32, LANES)
    wid = sid * 2 + cid
    base = wid * 512

    def chunk(ci, _):
        off = base + ci * 128
        pltpu.sync_copy(h_hbm.at[pl.ds(off, 128)], hi)
        pltpu.sync_copy(t_hbm.at[pl.ds(off, 128)], ti)
        pltpu.sync_copy(r_hbm.at[pl.ds(off, 128)], ri)
        cp1 = pltpu.async_copy(ent_hbm.at[hi], hb, sem)
        cp2 = pltpu.async_copy(ent_hbm.at[ti], tb2, sem)
        cp3 = pltpu.async_copy(rel_hbm.at[ri], rb, sem)
        cp1.wait()
        cp2.wait()
        cp3.wait()

        def grp(rg, _):
            rows = rg * LANES + lanes
            l8 = lanes & 7

            def kbody(k, accs):
                col0 = k * 8
                new = []
                for j in range(8):
                    cols = ((l8 + j) & 7) + col0
                    hv = plsc.load_gather(hb, [rows, cols])
                    tv = plsc.load_gather(tb2, [rows, cols])
                    rv = plsc.load_gather(rb, [rows, cols])
                    new.append(accs[j] + jnp.abs(hv + rv - tv))
                return tuple(new)

            accs = lax.fori_loop(
                0, 16, kbody,
                tuple(jnp.zeros((LANES,), jnp.float32) for _ in range(8)))
            q0 = accs[0] + accs[4]
            q1 = accs[1] + accs[5]
            q2 = accs[2] + accs[6]
            q3 = accs[3] + accs[7]
            sb[pl.ds(rg * LANES, LANES)] = -((q0 + q2) + (q1 + q3))
            return 0

        lax.fori_loop(0, 8, grp, 0)
        pltpu.sync_copy(sb, ps_hbm.at[pl.ds(off, 128)])
        return 0

    lax.fori_loop(0, 4, chunk, 0)


def _sc_score(h_idx, t_idx, r_idx, ent_table, rel_table):
    mesh = plsc.VectorSubcoreMesh(core_axis_name="c", subcore_axis_name="s")
    return pl.kernel(
        _score_sc_body,
        mesh=mesh,
        compiler_params=pltpu.CompilerParams(needs_layout_passes=False),
        out_type=jax.ShapeDtypeStruct((BATCH,), jnp.float32),
        scratch_types=[
            pltpu.VMEM((128,), jnp.int32),
            pltpu.VMEM((128,), jnp.int32),
            pltpu.VMEM((128,), jnp.int32),
            pltpu.VMEM((128, EMBED), jnp.float32),
            pltpu.VMEM((128, EMBED), jnp.float32),
            pltpu.VMEM((128, EMBED), jnp.float32),
            pltpu.VMEM((128,), jnp.float32),
            pltpu.SemaphoreType.DMA,
        ],
    )(h_idx, t_idx, r_idx, ent_table, rel_table)


def _sort_body(score_hbm, out_hbm, sb, kb, vb, lh, l0, cnt, tots, gt, cb,
               tb, pc, ko, vo, po, SK, SV, STOT, sem):
    cid = lax.axis_index("c")
    T = lax.axis_index("s")
    lanes = lax.iota(jnp.int32, LANES)
    U = 4
    NPASS = 6

    @pl.when(cid == 0)
    def _():
        base = T * 1024
        pltpu.sync_copy(score_hbm.at[pl.ds(base, 1024)], sb)

        def build(g, _):
            s = sb[pl.ds(g * LANES, LANES)]
            b = lax.bitcast_convert_type(-s, jnp.int32)
            key = jnp.where(b < 0, ~b, b | jnp.int32(-2**31))
            kb[pl.ds(g * LANES, LANES)] = key
            vb[pl.ds(g * LANES, LANES)] = base + g * LANES + lanes
            return 0

        lax.fori_loop(0, 64, build, 0)
        pltpu.sync_copy(kb, SK.at[pl.ds(base, 1024)])
        pltpu.sync_copy(vb, SV.at[pl.ds(base, 1024)])
        plsc.subcore_barrier()

        for p in range(NPASS):
            src_off = (p % 2) * BATCH
            dst_off = BATCH - src_off
            shift = jnp.full((LANES,), 6 * p, jnp.int32)
            pltpu.sync_copy(SK.at[pl.ds(src_off + base, 1024)], kb)
            pltpu.sync_copy(SV.at[pl.ds(src_off + base, 1024)], vb)

            def zero(i, _):
                for u in range(U):
                    lh[pl.ds((i * U + u) * LANES, LANES)] = jnp.zeros(
                        (LANES,), jnp.int32)
                return 0

            lax.fori_loop(0, 64 // U, zero, 0)

            def histo(i, _, shift=shift):
                idxs = []
                for u in range(U):
                    key = plsc.load_gather(kb, [lanes * 64 + (i * U + u)])
                    d = jnp.bitwise_and(
                        lax.shift_right_logical(key, shift), 63)
                    idxs.append(d * LANES + lanes)
                cs = [plsc.load_gather(lh, [idxs[u]]) for u in range(U)]
                for u in range(U):
                    c = cs[u]
                    for w in range(u):
                        c = c + jnp.where(idxs[u] == idxs[w], 1, 0)
                    plsc.store_scatter(lh, [idxs[u]], c + 1)
                return 0

            lax.fori_loop(0, 64 // U, histo, 0)

            def scan(i, carry):
                v = lh[pl.ds(i * LANES, LANES)]
                inc = plsc.cumsum(v)
                lh[pl.ds(i * LANES, LANES)] = inc - v + carry
                return carry + jnp.sum(v)

            lax.fori_loop(0, 64, scan, jnp.int32(0))

            def ext(i, _):
                dg = i * LANES + lanes
                l0[pl.ds(i * LANES, LANES)] = plsc.load_gather(lh, [dg * LANES])
                return 0

            lax.fori_loop(0, 4, ext, 0)
            l0[pl.ds(64, LANES)] = jnp.full((LANES,), 1024, jnp.int32)

            def cnts(i, _):
                cur = l0[pl.ds(i * LANES, LANES)]
                nxt = plsc.load_gather(l0, [i * LANES + lanes + 1])
                cnt[pl.ds(i * LANES, LANES)] = nxt - cur
                return 0

            lax.fori_loop(0, 4, cnts, 0)
            pltpu.sync_copy(cnt, STOT.at[pl.ds(T * 64, 64)])
            plsc.subcore_barrier()
            pltpu.sync_copy(STOT, tots)
            if _DBG == "stot":
                pltpu.sync_copy(tots, out_hbm.at[pl.ds(base, 1024)])
                return

            def cg(i, _):
                acc_t = jnp.zeros((LANES,), jnp.int32)
                acc_c = jnp.zeros((LANES,), jnp.int32)
                for Tp in range(16):
                    v = tots[pl.ds(Tp * 64 + i * LANES, LANES)]
                    acc_t = acc_t + v
                    m = jnp.where(jnp.int32(Tp) < T, 1, 0)
                    acc_c = acc_c + v * m
                gt[pl.ds(i * LANES, LANES)] = acc_t
                cb[pl.ds(i * LANES, LANES)] = acc_c
                return 0

            lax.fori_loop(0, 4, cg, 0)
            if _DBG == "gt":
                pltpu.sync_copy(gt, out_hbm.at[pl.ds(base, 64)])

            def gscan(i, carry):
                v = gt[pl.ds(i * LANES, LANES)]
                inc = plsc.cumsum(v)
                gt[pl.ds(i * LANES, LANES)] = inc - v + carry
                return carry + jnp.sum(v)

            lax.fori_loop(0, 4, gscan, jnp.int32(0))
            if _DBG == "gt":
                pltpu.sync_copy(gt, out_hbm.at[pl.ds(base + 64, 64)])
                pltpu.sync_copy(cb, out_hbm.at[pl.ds(base + 128, 64)])
                pltpu.sync_copy(cnt, out_hbm.at[pl.ds(base + 192, 64)])
                return

            def pgrp(i, _):
                dg = i * LANES + lanes
                offv = (plsc.load_gather(gt, [dg]) + plsc.load_gather(cb, [dg])
                        - plsc.load_gather(l0, [dg]))
                tb[pl.ds(0, LANES)] = offv

                def pinner(j, _):
                    off_b = plsc.load_gather(
                        tb, [jnp.full((LANES,), 0, jnp.int32) + j])
                    d16 = (i * LANES + j) * LANES
                    pc[pl.ds(d16, LANES)] = lh[pl.ds(d16, LANES)] + off_b
                    return 0

                lax.fori_loop(0, LANES, pinner, 0)
                return 0

            lax.fori_loop(0, 4, pgrp, 0)
            if _DBG == "pc":
                pltpu.sync_copy(pc, out_hbm.at[pl.ds(base, 1024)])

            def perm(i, _, shift=shift, dst_off=dst_off):
                row = i // 2
                colb = (i % 2) * 64
                keys = []
                vals = []
                ids = []
                for u in range(U):
                    e = lanes * 64 + (i * U + u)
                    keys.append(plsc.load_gather(kb, [e]))
                    vals.append(plsc.load_gather(vb, [e]))
                    d = jnp.bitwise_and(
                        lax.shift_right_logical(keys[u], shift), 63)
                    ids.append(d * LANES + lanes)
                ps = [plsc.load_gather(pc, [ids[u]]) for u in range(U)]
                for u in range(U):
                    pos = ps[u]
                    for w in range(u):
                        pos = pos + jnp.where(ids[u] == ids[w], 1, 0)
                    plsc.store_scatter(pc, [ids[u]], pos + 1)
                    ko[row, pl.ds(colb + u * LANES, LANES)] = keys[u]
                    vo[row, pl.ds(colb + u * LANES, LANES)] = vals[u]
                    po[row, pl.ds(colb + u * LANES, LANES)] = pos + dst_off
                return 0

            if _DBG == "pc":
                return
            lax.fori_loop(0, 64 // U, perm, 0)
            cps = []
            for j in range(8):
                cps.append(pltpu.async_copy(ko.at[j], SK.at[po.at[j]], sem))
                cps.append(pltpu.async_copy(vo.at[j], SV.at[po.at[j]], sem))
            for c in cps:
                c.wait()
            plsc.subcore_barrier()

        fin = (NPASS % 2) * BATCH
        pltpu.sync_copy(SV.at[pl.ds(fin + base, 1024)], vb)
        pltpu.sync_copy(vb, out_hbm.at[pl.ds(base, 1024)])


def _sc_sort(p_score):
    mesh = plsc.VectorSubcoreMesh(core_axis_name="c", subcore_axis_name="s")
    return pl.kernel(
        _sort_body,
        mesh=mesh,
        compiler_params=pltpu.CompilerParams(needs_layout_passes=False),
        out_type=jax.ShapeDtypeStruct((BATCH,), jnp.int32),
        scratch_types=[
            pltpu.VMEM((1024,), jnp.float32),
            pltpu.VMEM((1024,), jnp.int32),
            pltpu.VMEM((1024,), jnp.int32),
            pltpu.VMEM((1024,), jnp.int32),
            pltpu.VMEM((80,), jnp.int32),
            pltpu.VMEM((64,), jnp.int32),
            pltpu.VMEM((1024,), jnp.int32),
            pltpu.VMEM((64,), jnp.int32),
            pltpu.VMEM((64,), jnp.int32),
            pltpu.VMEM((64,), jnp.int32),
            pltpu.VMEM((1024,), jnp.int32),
            pltpu.VMEM((8, 128), jnp.int32),
            pltpu.VMEM((8, 128), jnp.int32),
            pltpu.VMEM((8, 128), jnp.int32),
            pltpu.VMEM_SHARED((2 * BATCH,), jnp.int32),
            pltpu.VMEM_SHARED((2 * BATCH,), jnp.int32),
            pltpu.VMEM_SHARED((1024,), jnp.int32),
            pltpu.SemaphoreType.DMA,
        ],
    )(p_score)


def kernel(h_idx, t_idx, r_idx, ent_table, rel_table):
    p_score = _sc_score(h_idx, t_idx, r_idx, ent_table, rel_table)
    ranked = _sc_sort(p_score)
    return (p_score, ranked)


# speedup vs baseline: 4.0657x; 1.0015x over previous
import jax
import jax.numpy as jnp
from jax import lax
from jax.experimental import pallas as pl
from jax.experimental.pallas import tpu as pltpu
from jax.experimental.pallas import tpu_sc as plsc

BATCH = 16384
EMBED = 128
LANES = 16


def _score_sc_body(h_hbm, t_hbm, r_hbm, ent_hbm, rel_hbm, ps_hbm,
                   hi, ti, ri, hb, tb2, rb, sb, sem):
    cid = lax.axis_index("c")
    sid = lax.axis_index("s")
    lanes = lax.iota(jnp.int32, LANES)
    wid = sid * 2 + cid
    base = wid * 512

    def chunk(ci, _):
        off = base + ci * 128
        pltpu.sync_copy(h_hbm.at[pl.ds(off, 128)], hi)
        pltpu.sync_copy(t_hbm.at[pl.ds(off, 128)], ti)
        pltpu.sync_copy(r_hbm.at[pl.ds(off, 128)], ri)
        cp1 = pltpu.async_copy(ent_hbm.at[hi], hb, sem)
        cp2 = pltpu.async_copy(ent_hbm.at[ti], tb2, sem)
        cp3 = pltpu.async_copy(rel_hbm.at[ri], rb, sem)
        cp1.wait()
        cp2.wait()
        cp3.wait()

        def grp(rg, _):
            rows = rg * LANES + lanes
            l8 = lanes & 7

            def kbody(k, accs):
                col0 = k * 8
                new = []
                for j in range(8):
                    cols = ((l8 + j) & 7) + col0
                    hv = plsc.load_gather(hb, [rows, cols])
                    tv = plsc.load_gather(tb2, [rows, cols])
                    rv = plsc.load_gather(rb, [rows, cols])
                    new.append(accs[j] + jnp.abs(hv + rv - tv))
                return tuple(new)

            accs = lax.fori_loop(
                0, 16, kbody,
                tuple(jnp.zeros((LANES,), jnp.float32) for _ in range(8)))
            q0 = accs[0] + accs[4]
            q1 = accs[1] + accs[5]
            q2 = accs[2] + accs[6]
            q3 = accs[3] + accs[7]
            sb[pl.ds(rg * LANES, LANES)] = -((q0 + q2) + (q1 + q3))
            return 0

        lax.fori_loop(0, 8, grp, 0)
        pltpu.sync_copy(sb, ps_hbm.at[pl.ds(off, 128)])
        return 0

    lax.fori_loop(0, 4, chunk, 0)


def _sc_score(h_idx, t_idx, r_idx, ent_table, rel_table):
    mesh = plsc.VectorSubcoreMesh(core_axis_name="c", subcore_axis_name="s")
    return pl.kernel(
        _score_sc_body,
        mesh=mesh,
        compiler_params=pltpu.CompilerParams(needs_layout_passes=False),
        out_type=jax.ShapeDtypeStruct((BATCH,), jnp.float32),
        scratch_types=[
            pltpu.VMEM((128,), jnp.int32),
            pltpu.VMEM((128,), jnp.int32),
            pltpu.VMEM((128,), jnp.int32),
            pltpu.VMEM((128, EMBED), jnp.float32),
            pltpu.VMEM((128, EMBED), jnp.float32),
            pltpu.VMEM((128, EMBED), jnp.float32),
            pltpu.VMEM((128,), jnp.float32),
            pltpu.SemaphoreType.DMA,
        ],
    )(h_idx, t_idx, r_idx, ent_table, rel_table)


def _sort_body(score_hbm, out_hbm, sb, kb, vb, lh, l0, cnt, tots, gt, cb,
               tb, pc, ko, vo, po, SK, SV, STOT, sem):
    cid = lax.axis_index("c")
    T = lax.axis_index("s")
    lanes = lax.iota(jnp.int32, LANES)
    U = 4
    NPASS = 6

    @pl.when(cid == 0)
    def _():
        base = T * 1024
        pltpu.sync_copy(score_hbm.at[pl.ds(base, 1024)], sb)

        def build(g, _):
            s = sb[pl.ds(g * LANES, LANES)]
            b = lax.bitcast_convert_type(-s, jnp.int32)
            key = jnp.where(b < 0, ~b, b | jnp.int32(-2**31))
            kb[pl.ds(g * LANES, LANES)] = key
            vb[pl.ds(g * LANES, LANES)] = base + g * LANES + lanes
            return 0

        lax.fori_loop(0, 64, build, 0)
        pltpu.sync_copy(kb, SK.at[pl.ds(base, 1024)])
        pltpu.sync_copy(vb, SV.at[pl.ds(base, 1024)])
        plsc.subcore_barrier()

        for p in range(NPASS):
            src_off = (p % 2) * BATCH
            dst_off = BATCH - src_off
            shift = jnp.full((LANES,), 6 * p, jnp.int32)
            pltpu.sync_copy(SK.at[pl.ds(src_off + base, 1024)], kb)
            pltpu.sync_copy(SV.at[pl.ds(src_off + base, 1024)], vb)

            def zero(i, _):
                for u in range(U):
                    lh[pl.ds((i * U + u) * LANES, LANES)] = jnp.zeros(
                        (LANES,), jnp.int32)
                return 0

            lax.fori_loop(0, 64 // U, zero, 0)

            def histo(i, _, shift=shift):
                idxs = []
                for u in range(U):
                    key = plsc.load_gather(kb, [lanes * 64 + (i * U + u)])
                    d = jnp.bitwise_and(
                        lax.shift_right_logical(key, shift), 63)
                    idxs.append(d * LANES + lanes)
                cs = [plsc.load_gather(lh, [idxs[u]]) for u in range(U)]
                for u in range(U):
                    c = cs[u]
                    for w in range(u):
                        c = c + jnp.where(idxs[u] == idxs[w], 1, 0)
                    plsc.store_scatter(lh, [idxs[u]], c + 1)
                return 0

            lax.fori_loop(0, 64 // U, histo, 0)

            def scan(i, carry):
                v = lh[pl.ds(i * LANES, LANES)]
                inc = plsc.cumsum(v)
                lh[pl.ds(i * LANES, LANES)] = inc - v + carry
                return carry + jnp.sum(v)

            lax.fori_loop(0, 64, scan, jnp.int32(0))

            def ext(i, _):
                dg = i * LANES + lanes
                l0[pl.ds(i * LANES, LANES)] = plsc.load_gather(lh, [dg * LANES])
                return 0

            lax.fori_loop(0, 4, ext, 0)
            l0[pl.ds(64, LANES)] = jnp.full((LANES,), 1024, jnp.int32)

            def cnts(i, _):
                cur = l0[pl.ds(i * LANES, LANES)]
                nxt = plsc.load_gather(l0, [i * LANES + lanes + 1])
                cnt[pl.ds(i * LANES, LANES)] = nxt - cur
                return 0

            lax.fori_loop(0, 4, cnts, 0)
            pltpu.sync_copy(cnt, STOT.at[pl.ds(T * 64, 64)])
            plsc.subcore_barrier()
            pltpu.sync_copy(STOT, tots)

            def cg(i, _):
                acc_t = jnp.zeros((LANES,), jnp.int32)
                acc_c = jnp.zeros((LANES,), jnp.int32)
                for Tp in range(16):
                    v = tots[pl.ds(Tp * 64 + i * LANES, LANES)]
                    acc_t = acc_t + v
                    m = jnp.where(jnp.int32(Tp) < T, 1, 0)
                    acc_c = acc_c + v * m
                gt[pl.ds(i * LANES, LANES)] = acc_t
                cb[pl.ds(i * LANES, LANES)] = acc_c
                return 0

            lax.fori_loop(0, 4, cg, 0)

            def gscan(i, carry):
                v = gt[pl.ds(i * LANES, LANES)]
                inc = plsc.cumsum(v)
                gt[pl.ds(i * LANES, LANES)] = inc - v + carry
                return carry + jnp.sum(v)

            lax.fori_loop(0, 4, gscan, jnp.int32(0))

            def pgrp(i, _):
                dg = i * LANES + lanes
                offv = (plsc.load_gather(gt, [dg]) + plsc.load_gather(cb, [dg])
                        - plsc.load_gather(l0, [dg]))
                tb[pl.ds(0, LANES)] = offv

                def pinner(j, _):
                    off_b = plsc.load_gather(
                        tb, [jnp.full((LANES,), 0, jnp.int32) + j])
                    d16 = (i * LANES + j) * LANES
                    pc[pl.ds(d16, LANES)] = lh[pl.ds(d16, LANES)] + off_b
                    return 0

                lax.fori_loop(0, LANES, pinner, 0)
                return 0

            lax.fori_loop(0, 4, pgrp, 0)

            def perm(i, _, shift=shift, dst_off=dst_off):
                row = i // 2
                colb = (i % 2) * 64
                keys = []
                vals = []
                ids = []
                for u in range(U):
                    e = lanes * 64 + (i * U + u)
                    keys.append(plsc.load_gather(kb, [e]))
                    vals.append(plsc.load_gather(vb, [e]))
                    d = jnp.bitwise_and(
                        lax.shift_right_logical(keys[u], shift), 63)
                    ids.append(d * LANES + lanes)
                ps = [plsc.load_gather(pc, [ids[u]]) for u in range(U)]
                for u in range(U):
                    pos = ps[u]
                    for w in range(u):
                        pos = pos + jnp.where(ids[u] == ids[w], 1, 0)
                    plsc.store_scatter(pc, [ids[u]], pos + 1)
                    ko[row, pl.ds(colb + u * LANES, LANES)] = keys[u]
                    vo[row, pl.ds(colb + u * LANES, LANES)] = vals[u]
                    po[row, pl.ds(colb + u * LANES, LANES)] = pos + dst_off
                return 0

            lax.fori_loop(0, 64 // U, perm, 0)
            cps = []
            for j in range(8):
                cps.append(pltpu.async_copy(ko.at[j], SK.at[po.at[j]], sem))
                cps.append(pltpu.async_copy(vo.at[j], SV.at[po.at[j]], sem))
            for c in cps:
                c.wait()
            plsc.subcore_barrier()

        pltpu.sync_copy(SV.at[pl.ds(base, 1024)], vb)
        pltpu.sync_copy(vb, out_hbm.at[pl.ds(base, 1024)])


def _sc_sort(p_score):
    mesh = plsc.VectorSubcoreMesh(core_axis_name="c", subcore_axis_name="s")
    return pl.kernel(
        _sort_body,
        mesh=mesh,
        compiler_params=pltpu.CompilerParams(needs_layout_passes=False),
        out_type=jax.ShapeDtypeStruct((BATCH,), jnp.int32),
        scratch_types=[
            pltpu.VMEM((1024,), jnp.float32),
            pltpu.VMEM((1024,), jnp.int32),
            pltpu.VMEM((1024,), jnp.int32),
            pltpu.VMEM((1024,), jnp.int32),
            pltpu.VMEM((80,), jnp.int32),
            pltpu.VMEM((64,), jnp.int32),
            pltpu.VMEM((1024,), jnp.int32),
            pltpu.VMEM((64,), jnp.int32),
            pltpu.VMEM((64,), jnp.int32),
            pltpu.VMEM((64,), jnp.int32),
            pltpu.VMEM((1024,), jnp.int32),
            pltpu.VMEM((8, 128), jnp.int32),
            pltpu.VMEM((8, 128), jnp.int32),
            pltpu.VMEM((8, 128), jnp.int32),
            pltpu.VMEM_SHARED((2 * BATCH,), jnp.int32),
            pltpu.VMEM_SHARED((2 * BATCH,), jnp.int32),
            pltpu.VMEM_SHARED((1024,), jnp.int32),
            pltpu.SemaphoreType.DMA,
        ],
    )(p_score)


def kernel(h_idx, t_idx, r_idx, ent_table, rel_table):
    p_score = _sc_score(h_idx, t_idx, r_idx, ent_table, rel_table)
    ranked = _sc_sort(p_score)
    return (p_score, ranked)
